# Initial kernel scaffold; baseline (speedup 1.0000x reference)
import jax
import jax.numpy as jnp
from jax.experimental import pallas as pl


def kernel(x, W):
    raise NotImplementedError("write your pallas kernel here")


# speedup vs baseline: 4.8816x; 4.8816x over previous
import functools

import jax
import jax.numpy as jnp
from jax import lax
from jax.experimental import pallas as pl
from jax.experimental.pallas import tpu as pltpu
from jax.experimental.pallas import tpu_sc as plsc

BATCH = 16384
SEQ = 200
NVAL = 10
DIM = 5
N = BATCH * SEQ
NC = 2
NS = 16
NW = NC * NS
PER_W = N // NW
CHUNK = 4096
NCHUNK = PER_W // CHUNK
VECS = CHUNK // 16
TSIZE = NVAL * DIM
TBL = 128

_mesh = plsc.VectorSubcoreMesh(core_axis_name="c", subcore_axis_name="s")


@functools.partial(
    pl.kernel,
    out_type=jax.ShapeDtypeStruct((N * DIM,), jnp.float32),
    mesh=_mesh,
    compiler_params=pltpu.CompilerParams(needs_layout_passes=False),
    scratch_types=[
        pltpu.VMEM((TBL,), jnp.float32),
        pltpu.VMEM((64,), jnp.float32),
        pltpu.VMEM((16,), jnp.float32),
        pltpu.VMEM((16,), jnp.int32),
        pltpu.VMEM((CHUNK,), jnp.int32),
        pltpu.VMEM((CHUNK * DIM,), jnp.float32),
    ],
)
def _sc_lookup(x_hbm, w_hbm, out_hbm, tbl_v, wtmp, scale_v, x0tmp, idx_v, out_v):
    wid = lax.axis_index("s") * NC + lax.axis_index("c")
    lanes = lax.iota(jnp.int32, 16)

    pltpu.sync_copy(w_hbm, wtmp)
    pltpu.sync_copy(x_hbm.at[pl.ds(0, 16)], x0tmp)

    x16 = x0tmp[...]
    x00 = jnp.sum(jnp.where(lanes == 0, x16, 0))
    waddr = jnp.minimum(x00 * DIM + lanes, TSIZE - 1)
    row = plsc.load_gather(wtmp, [waddr])
    valid = lanes < DIM
    rowm = jnp.where(valid, row, jnp.float32(-1e30))
    m = jnp.max(rowm)
    e = jnp.where(valid, jnp.exp(rowm - m), jnp.float32(0.0))
    scale_v[...] = e / jnp.sum(e)

    for k in range(TBL // 16):
        j = lanes + k * 16
        jm = jnp.where(j < TSIZE, j, j - TSIZE)
        jm = jnp.minimum(jm, TSIZE - 1)
        wv = plsc.load_gather(wtmp, [jm])
        sv = plsc.load_gather(scale_v, [jm % DIM])
        tbl_v[pl.ds(k * 16, 16)] = jnp.where(j < TSIZE, wv, wv * sv)

    base = wid * PER_W

    def chunk_body(ci, carry):
        gbase = base + ci * CHUNK
        pltpu.sync_copy(x_hbm.at[pl.ds(gbase, CHUNK)], idx_v)

        def vec_body(i, c2):
            lpos = i * 16
            idx = idx_v[pl.ds(lpos, 16)]
            gpos = gbase + lpos + lanes
            is0 = (gpos % SEQ) == 0
            addr = idx * DIM + jnp.where(is0, TSIZE, 0)
            opos = lpos * DIM + lanes * DIM
            for c in range(DIM):
                vals = plsc.load_gather(tbl_v, [addr + c])
                plsc.store_scatter(out_v, [opos + c], vals)
            return c2

        lax.fori_loop(0, VECS, vec_body, 0)
        pltpu.sync_copy(out_v, out_hbm.at[pl.ds(gbase * DIM, CHUNK * DIM)])
        return carry

    lax.fori_loop(0, NCHUNK, chunk_body, 0)


def kernel(x, W):
    xf = x.reshape(-1)
    wf = jnp.pad(W.reshape(-1), (0, 64 - TSIZE))
    out = _sc_lookup(xf, wf)
    return out.reshape(BATCH, SEQ, DIM)


# speedup vs baseline: 63.1443x; 12.9351x over previous
import functools

import jax
import jax.numpy as jnp
from jax import lax
from jax.experimental import pallas as pl
from jax.experimental.pallas import tpu as pltpu
from jax.experimental.pallas import tpu_sc as plsc

BATCH = 16384
SEQ = 200
NVAL = 10
DIM = 5
NC = 2
NS = 16
NW = NC * NS
WCOLS = BATCH // NW
NSTRIP = SEQ // 8
TSIZE = NVAL * DIM
TBL = 128

_mesh = plsc.VectorSubcoreMesh(core_axis_name="c", subcore_axis_name="s")


@functools.partial(
    pl.kernel,
    out_type=jax.ShapeDtypeStruct((DIM, SEQ, BATCH), jnp.float32),
    mesh=_mesh,
    compiler_params=pltpu.CompilerParams(needs_layout_passes=False),
    scratch_types=[
        pltpu.VMEM((TBL,), jnp.float32),
        pltpu.VMEM((64,), jnp.float32),
        pltpu.VMEM((16,), jnp.float32),
        pltpu.VMEM((16,), jnp.int32),
        pltpu.VMEM((8, WCOLS), jnp.int32),
        pltpu.VMEM((DIM, 8, WCOLS), jnp.float32),
    ],
)
def _sc_lookup(xt_hbm, w_hbm, out_hbm, tbl_v, wtmp, scale_v, x0tmp, xtile, otile):
    wid = lax.axis_index("s") * NC + lax.axis_index("c")
    lanes = lax.iota(jnp.int32, 16)

    pltpu.sync_copy(w_hbm, wtmp)
    pltpu.sync_copy(xt_hbm.at[0, pl.ds(0, 16)], x0tmp)

    x16 = x0tmp[...]
    x00 = jnp.sum(jnp.where(lanes == 0, x16, 0))
    waddr = jnp.minimum(x00 * DIM + lanes, TSIZE - 1)
    row = plsc.load_gather(wtmp, [waddr])
    valid = lanes < DIM
    rowm = jnp.where(valid, row, jnp.float32(-1e30))
    m = jnp.max(rowm)
    e = jnp.where(valid, jnp.exp(rowm - m), jnp.float32(0.0))
    scale_v[...] = e / jnp.sum(e)

    for k in range(TBL // 16):
        j = lanes + k * 16
        jm = jnp.where(j < TSIZE, j, j - TSIZE)
        jm = jnp.minimum(jm, TSIZE - 1)
        wv = plsc.load_gather(wtmp, [jm])
        sv = plsc.load_gather(scale_v, [jm % DIM])
        tbl_v[pl.ds(k * 16, 16)] = jnp.where(j < TSIZE, wv, wv * sv)

    col0 = wid * WCOLS

    def strip_body(t, carry):
        pltpu.sync_copy(xt_hbm.at[pl.ds(t * 8, 8), pl.ds(col0, WCOLS)], xtile)
        row0_extra = jnp.where(t == 0, TSIZE, 0)

        def vec_body(i, c2):
            o = i * 16
            for r in range(8):
                idx = xtile[r, pl.ds(o, 16)]
                addr = idx * DIM
                if r == 0:
                    addr = addr + row0_extra
                for c in range(DIM):
                    otile[c, r, pl.ds(o, 16)] = plsc.load_gather(tbl_v, [addr + c])
            return c2

        lax.fori_loop(0, WCOLS // 16, vec_body, 0)
        for c in range(DIM):
            pltpu.sync_copy(otile.at[c],
                            out_hbm.at[c, pl.ds(t * 8, 8), pl.ds(col0, WCOLS)])
        return carry

    lax.fori_loop(0, NSTRIP, strip_body, 0)


def kernel(x, W):
    xt = x.T
    wf = jnp.pad(W.reshape(-1), (0, 64 - TSIZE))
    out = _sc_lookup(xt, wf)
    return out.transpose(2, 1, 0)


# speedup vs baseline: 220.8548x; 3.4976x over previous
import functools

import jax
import jax.numpy as jnp
from jax import lax
from jax.experimental import pallas as pl
from jax.experimental.pallas import tpu as pltpu
from jax.experimental.pallas import tpu_sc as plsc

BATCH = 16384
SEQ = 200
NVAL = 10
DIM = 5
NC = 2
NS = 16
NW = NC * NS
WCOLS = BATCH // NW
NSTRIP = SEQ // 8
TSIZE = NVAL * DIM
TBL = 128

_mesh = plsc.VectorSubcoreMesh(core_axis_name="c", subcore_axis_name="s")


@functools.partial(
    pl.kernel,
    out_type=jax.ShapeDtypeStruct((DIM, SEQ, BATCH), jnp.float32),
    mesh=_mesh,
    compiler_params=pltpu.CompilerParams(needs_layout_passes=False),
    scratch_types=[
        pltpu.VMEM((TBL,), jnp.float32),
        pltpu.VMEM((64,), jnp.float32),
        pltpu.VMEM((16,), jnp.float32),
        pltpu.VMEM((16,), jnp.int32),
        pltpu.VMEM((2, 8, WCOLS), jnp.int32),
        pltpu.VMEM((2, DIM, 8, WCOLS), jnp.float32),
        pltpu.SemaphoreType.DMA((2,)),
        pltpu.SemaphoreType.DMA((2,)),
    ],
)
def _sc_lookup(xt_hbm, w_hbm, out_hbm, tbl_v, wtmp, scale_v, x0tmp, xtile, otile,
               sem_in, sem_out):
    wid = lax.axis_index("s") * NC + lax.axis_index("c")
    lanes = lax.iota(jnp.int32, 16)

    pltpu.sync_copy(w_hbm, wtmp)
    pltpu.sync_copy(xt_hbm.at[0, pl.ds(0, 16)], x0tmp)

    x16 = x0tmp[...]
    x00 = jnp.sum(jnp.where(lanes == 0, x16, 0))
    waddr = jnp.minimum(x00 * DIM + lanes, TSIZE - 1)
    row = plsc.load_gather(wtmp, [waddr])
    valid = lanes < DIM
    rowm = jnp.where(valid, row, jnp.float32(-1e30))
    m = jnp.max(rowm)
    e = jnp.where(valid, jnp.exp(rowm - m), jnp.float32(0.0))
    scale_v[...] = e / jnp.sum(e)

    for k in range(TBL // 16):
        j = lanes + k * 16
        jm = jnp.where(j < TSIZE, j, j - TSIZE)
        jm = jnp.minimum(jm, TSIZE - 1)
        wv = plsc.load_gather(wtmp, [jm])
        sv = plsc.load_gather(scale_v, [jm % DIM])
        tbl_v[pl.ds(k * 16, 16)] = jnp.where(j < TSIZE, wv, wv * sv)

    col0 = wid * WCOLS

    def in_copy(t, b):
        return pltpu.make_async_copy(
            xt_hbm.at[pl.ds(t * 8, 8), pl.ds(col0, WCOLS)],
            xtile.at[b], sem_in.at[b])

    def out_copy(t, b, c):
        return pltpu.make_async_copy(
            otile.at[b, c], out_hbm.at[c, pl.ds(t * 8, 8), pl.ds(col0, WCOLS)],
            sem_out.at[b])

    in_copy(0, 0).start()

    def strip_body(t, carry):
        b = t & 1

        @pl.when(t + 1 < NSTRIP)
        def _prefetch():
            in_copy(t + 1, 1 - b).start()

        in_copy(t, b).wait()

        @pl.when(t >= 2)
        def _drain_prev():
            for c in range(DIM):
                out_copy(t - 2, b, c).wait()

        row0_extra = jnp.where(t == 0, TSIZE, 0)

        @plsc.parallel_loop(0, WCOLS // 16, unroll=2)
        def vec_body(i):
            o = i * 16
            for r in range(8):
                idx = xtile[b, r, pl.ds(o, 16)]
                addr = idx * DIM
                if r == 0:
                    addr = addr + row0_extra
                for c in range(DIM):
                    otile[b, c, r, pl.ds(o, 16)] = plsc.load_gather(
                        tbl_v, [addr + c])

        for c in range(DIM):
            out_copy(t, b, c).start()
        return carry

    lax.fori_loop(0, NSTRIP, strip_body, 0)
    for c in range(DIM):
        out_copy(NSTRIP - 2, 1, c).wait()
        out_copy(NSTRIP - 1, 0, c).wait()


def kernel(x, W):
    xt = x.T
    wf = jnp.pad(W.reshape(-1), (0, 64 - TSIZE))
    out = _sc_lookup(xt, wf)
    return out.transpose(2, 1, 0)


# speedup vs baseline: 251.0158x; 1.1366x over previous
import functools

import jax
import jax.numpy as jnp
from jax import lax
from jax.experimental import pallas as pl
from jax.experimental.pallas import tpu as pltpu
from jax.experimental.pallas import tpu_sc as plsc

BATCH = 16384
SEQ = 200
NVAL = 10
DIM = 5
NC = 2
NS = 16
NW = NC * NS
WCOLS = BATCH // NW
NSTRIP = SEQ // 8
TSIZE = NVAL * DIM
TBL = 128

_mesh = plsc.VectorSubcoreMesh(core_axis_name="c", subcore_axis_name="s")


@functools.partial(
    pl.kernel,
    out_type=jax.ShapeDtypeStruct((DIM, SEQ, BATCH), jnp.float32),
    mesh=_mesh,
    compiler_params=pltpu.CompilerParams(needs_layout_passes=False),
    scratch_types=[
        pltpu.VMEM((TBL,), jnp.float32),
        pltpu.VMEM((64,), jnp.float32),
        pltpu.VMEM((16,), jnp.float32),
        pltpu.VMEM((16,), jnp.int32),
        pltpu.VMEM((2, 8, WCOLS), jnp.int32),
        pltpu.VMEM((2, DIM, 8, WCOLS), jnp.float32),
        pltpu.SemaphoreType.DMA((2,)),
        pltpu.SemaphoreType.DMA((2,)),
    ],
)
def _sc_lookup(xt_hbm, w_hbm, out_hbm, tbl_v, wtmp, scale_v, x0tmp, xtile, otile,
               sem_in, sem_out):
    wid = lax.axis_index("s") * NC + lax.axis_index("c")
    lanes = lax.iota(jnp.int32, 16)

    pltpu.sync_copy(w_hbm, wtmp)
    pltpu.sync_copy(xt_hbm.at[0, pl.ds(0, 16)], x0tmp)

    x16 = x0tmp[...]
    x00 = jnp.sum(jnp.where(lanes == 0, x16, 0))
    waddr = jnp.minimum(x00 * DIM + lanes, TSIZE - 1)
    row = plsc.load_gather(wtmp, [waddr])
    valid = lanes < DIM
    rowm = jnp.where(valid, row, jnp.float32(-1e30))
    m = jnp.max(rowm)
    e = jnp.where(valid, jnp.exp(rowm - m), jnp.float32(0.0))
    scale_v[...] = e / jnp.sum(e)

    for k in range(TBL // 16):
        j = lanes + k * 16
        jm = jnp.where(j < TSIZE, j, j - TSIZE)
        jm = jnp.minimum(jm, TSIZE - 1)
        wv = plsc.load_gather(wtmp, [jm])
        sv = plsc.load_gather(scale_v, [jm % DIM])
        tbl_v[pl.ds(k * 16, 16)] = jnp.where(j < TSIZE, wv, wv * sv)

    luts = []
    for c in range(DIM):
        laddr = jnp.minimum(lanes * DIM + c, TSIZE - 1)
        luts.append(plsc.load_gather(wtmp, [laddr]))

    col0 = wid * WCOLS

    def in_copy(t, b):
        return pltpu.make_async_copy(
            xt_hbm.at[pl.ds(t * 8, 8), pl.ds(col0, WCOLS)],
            xtile.at[b], sem_in.at[b])

    def out_copy(t, b, c):
        return pltpu.make_async_copy(
            otile.at[b, c], out_hbm.at[c, pl.ds(t * 8, 8), pl.ds(col0, WCOLS)],
            sem_out.at[b])

    in_copy(0, 0).start()

    def strip_body(t, carry):
        b = t & 1

        @pl.when(t + 1 < NSTRIP)
        def _prefetch():
            in_copy(t + 1, 1 - b).start()

        in_copy(t, b).wait()

        @pl.when(t >= 2)
        def _drain_prev():
            for c in range(DIM):
                out_copy(t - 2, b, c).wait()

        row0_extra = jnp.where(t == 0, TSIZE, 0)

        @plsc.parallel_loop(0, WCOLS // 16, unroll=2)
        def vec_body(i):
            o = i * 16
            for r in range(8):
                idx = xtile[b, r, pl.ds(o, 16)]
                if r == 0:
                    addr = idx * DIM + row0_extra
                    for c in range(DIM):
                        otile[b, c, r, pl.ds(o, 16)] = plsc.load_gather(
                            tbl_v, [addr + c])
                else:
                    for c in range(DIM):
                        otile[b, c, r, pl.ds(o, 16)] = luts[c].at[idx].get(
                            mode="promise_in_bounds")

        for c in range(DIM):
            out_copy(t, b, c).start()
        return carry

    lax.fori_loop(0, NSTRIP, strip_body, 0)
    for c in range(DIM):
        out_copy(NSTRIP - 2, 1, c).wait()
        out_copy(NSTRIP - 1, 0, c).wait()


def kernel(x, W):
    xt = x.T
    wf = jnp.pad(W.reshape(-1), (0, 64 - TSIZE))
    out = _sc_lookup(xt, wf)
    return out.transpose(2, 1, 0)
